# baseline (speedup 1.0000x reference)
import functools
import math

import jax
import jax.numpy as jnp
from jax import lax
from jax.experimental import pallas as pl
from jax.experimental.pallas import tpu as pltpu
from jax.experimental.pallas import tpu_sc as plsc

N = 10000
C = 256
S = 64
H = 8
D = C // H
E = 160000
EPS = 1e-6

NC = 2
NS = 16
LANES = 16
NW = NC * NS
BE = 128
EPT = 5120
NB = EPT // BE
EPAD = NW * EPT
NBUF = 4
ZTILES = 10
ZROWS = N // ZTILES
OUT_STRIDE = 16000

R = 2000
G = N // R


def _tc1_body(x_ref, wst_ref, bs_ref, w_ref, xtw_ref, wsum_ref):
    i = pl.program_id(0)
    xb = x_ref[...]
    logits = lax.dot_general(xb, wst_ref[...], (((1,), (0,)), ((), ())),
                             preferred_element_type=jnp.float32)
    logits = logits + bs_ref[...]
    m = jnp.max(logits, axis=1, keepdims=True)
    ew = jnp.exp(logits - m)
    w = ew / jnp.sum(ew, axis=1, keepdims=True)
    w_ref[...] = w
    xtw = lax.dot_general(xb, w, (((0,), (0,)), ((), ())),
                          preferred_element_type=jnp.float32)
    ws = jnp.broadcast_to(jnp.sum(w, axis=0, keepdims=True), (8, S))

    @pl.when(i == 0)
    def _init():
        xtw_ref[...] = jnp.zeros_like(xtw_ref)
        wsum_ref[...] = jnp.zeros_like(wsum_ref)

    xtw_ref[...] += xtw
    wsum_ref[...] += ws


def _tc1(xs, wst, bs_row):
    return pl.pallas_call(
        _tc1_body,
        grid=(G,),
        in_specs=[
            pl.BlockSpec((R, C), lambda i: (i, 0)),
            pl.BlockSpec((C, S), lambda i: (0, 0)),
            pl.BlockSpec((1, S), lambda i: (0, 0)),
        ],
        out_specs=[
            pl.BlockSpec((R, S), lambda i: (i, 0)),
            pl.BlockSpec((C, S), lambda i: (0, 0)),
            pl.BlockSpec((8, S), lambda i: (0, 0)),
        ],
        out_shape=[
            jax.ShapeDtypeStruct((N, S), jnp.float32),
            jax.ShapeDtypeStruct((C, S), jnp.float32),
            jax.ShapeDtypeStruct((8, S), jnp.float32),
        ],
    )(xs, wst, bs_row)


def _sc_agg_body(w_hbm, src_hbm, dst_hbm, val_hbm, zro_hbm, out_hbm,
                 src_v, dst_v, val_v, rows0, rows1, rows2, rows3, acc_sh,
                 sg0, sg1, sg2, sg3, ss0, ss1, ss2, ss3):
    cid = lax.axis_index("c")
    sid = lax.axis_index("s")
    wid = cid * NS + sid
    bufs = ((rows0, sg0, ss0), (rows1, sg1, ss1),
            (rows2, sg2, ss2), (rows3, sg3, ss3))

    pltpu.sync_copy(src_hbm.at[wid], src_v)
    pltpu.sync_copy(dst_hbm.at[wid], dst_v)
    pltpu.sync_copy(val_hbm.at[wid], val_v)
    pltpu.async_copy(w_hbm.at[src_v.at[0]], rows0, sg0)
    pltpu.async_copy(w_hbm.at[src_v.at[1]], rows1, sg1)
    @pl.when(sid < ZTILES)
    def _zero():
        pltpu.sync_copy(zro_hbm.at[pl.ds(sid * ZROWS, ZROWS)],
                        acc_sh.at[pl.ds(sid * ZROWS, ZROWS)])

    plsc.subcore_barrier()

    def quad(q, carry):
        for b in range(NBUF):
            j = NBUF * q + b
            rows_v, sg, ss = bufs[b]
            pltpu.make_async_copy(w_hbm.at[src_v.at[j]], rows_v, sg).wait()

            @plsc.parallel_loop(0, BE // LANES, 1, unroll=4)
            def _scale(g):
                vals16 = val_v[j, pl.ds(g * LANES, LANES)]
                for el in range(LANES):
                    v = vals16[el]
                    e = g * LANES + el
                    for cc in range(S // LANES):
                        sl = pl.ds(cc * LANES, LANES)
                        rows_v[e, sl] = rows_v[e, sl] * v

            pltpu.async_copy(rows_v, acc_sh.at[dst_v.at[j]], ss, add=True)

            bp = (b + 2) % NBUF
            rows_p, sgp, ssp = bufs[bp]

            @pl.when(j + 2 < NB)
            def _refill():
                @pl.when(j >= 2)
                def _drain():
                    pltpu.make_async_copy(
                        rows_p, acc_sh.at[dst_v.at[j - 2]], ssp).wait()

                pltpu.async_copy(w_hbm.at[src_v.at[j + 2]], rows_p, sgp)
        return carry

    lax.fori_loop(0, NB // NBUF, quad, 0)
    for b in range(NBUF):
        rows_v, sg, ss = bufs[b]
        pltpu.make_async_copy(rows_v, acc_sh.at[dst_v.at[NB - NBUF + b]],
                              ss).wait()
    plsc.subcore_barrier()
    @pl.when(sid < ZTILES)
    def _out():
        pltpu.sync_copy(acc_sh.at[pl.ds(sid * ZROWS, ZROWS)],
                        out_hbm.at[pl.ds(cid * OUT_STRIDE + sid * ZROWS,
                                         ZROWS)])


def _sc_agg(weights, srcp, dstp, valp, zeros_t):
    mesh = plsc.VectorSubcoreMesh(core_axis_name="c", subcore_axis_name="s")
    f = functools.partial(
        pl.kernel,
        mesh=mesh,
        compiler_params=pltpu.CompilerParams(use_tc_tiling_on_sc=False),
        out_type=jax.ShapeDtypeStruct((NC * OUT_STRIDE, S), jnp.float32),
        scratch_types=(
            [pltpu.VMEM((NB, BE), jnp.int32),
             pltpu.VMEM((NB, BE), jnp.int32),
             pltpu.VMEM((NB, BE), jnp.float32)]
            + [pltpu.VMEM((BE, S), jnp.float32)] * NBUF
            + [pltpu.VMEM_SHARED((N, S), jnp.float32)]
            + [pltpu.SemaphoreType.DMA] * (2 * NBUF)
        ),
    )(_sc_agg_body)
    return f(weights, srcp, dstp, valp, zeros_t)


def _tc2_body(w_ref, g0_ref, g1_ref, xtw_ref, wsum_ref,
              wq_ref, wk_ref, wv_ref, wo_ref,
              bq_ref, bk_ref, bv_ref, bo_ref, beta_ref,
              so_ref, bias_acc, attn_acc):
    i = pl.program_id(0)
    w = w_ref[...]
    g = g0_ref[...] + g1_ref[...]
    part = lax.dot_general(w, g, (((0,), (0,)), ((), ())),
                           preferred_element_type=jnp.float32)

    @pl.when(i == 0)
    def _init():
        bias_acc[...] = jnp.zeros_like(bias_acc)

    bias_acc[...] += part

    @pl.when(i == G - 1)
    def _finish():
        wsum = jnp.maximum(wsum_ref[0:1, :], EPS)
        slices_t = xtw_ref[...] / wsum
        qt = lax.dot_general(wq_ref[...], slices_t, (((1,), (0,)), ((), ())),
                             preferred_element_type=jnp.float32) + bq_ref[...]
        kt = lax.dot_general(wk_ref[...], slices_t, (((1,), (0,)), ((), ())),
                             preferred_element_type=jnp.float32) + bk_ref[...]
        vt = lax.dot_general(wv_ref[...], slices_t, (((1,), (0,)), ((), ())),
                             preferred_element_type=jnp.float32) + bv_ref[...]

        bias = bias_acc[...]
        ii = lax.broadcasted_iota(jnp.int32, (S, S), 0)
        jj = lax.broadcasted_iota(jnp.int32, (S, S), 1)
        eye = (ii == jj).astype(jnp.float32)
        bias_tr = lax.dot_general(bias, eye, (((0,), (0,)), ((), ())),
                                  preferred_element_type=jnp.float32)
        gb = 0.5 * (bias + bias_tr)
        gb = gb / jnp.maximum(jnp.sum(gb, axis=1, keepdims=True), EPS)
        gb = jnp.log(jnp.maximum(gb, EPS))
        br = beta_ref[0, 0]
        beta = jnp.maximum(br, 0.0) + jnp.log(1.0 + jnp.exp(-jnp.abs(br)))
        gbias = beta * gb
        scale = 1.0 / math.sqrt(D)

        for h in range(H):
            qh = qt[h * D:(h + 1) * D, :]
            kh = kt[h * D:(h + 1) * D, :]
            vh = vt[h * D:(h + 1) * D, :]
            lg = lax.dot_general(qh, kh, (((0,), (0,)), ((), ())),
                                 preferred_element_type=jnp.float32)
            lg = lg * scale + gbias
            m = jnp.max(lg, axis=1, keepdims=True)
            a = jnp.exp(lg - m)
            a = a / jnp.sum(a, axis=1, keepdims=True)
            oh = lax.dot_general(vh, a, (((1,), (1,)), ((), ())),
                                 preferred_element_type=jnp.float32)
            attn_acc[h * D:(h + 1) * D, :] = oh

        so_ref[...] = lax.dot_general(
            wo_ref[...], attn_acc[...], (((1,), (0,)), ((), ())),
            preferred_element_type=jnp.float32) + bo_ref[...]


def _tc2(weights, g0, g1, xtw, wsum, wq, wk, wv, wo,
         bq_m, bk_m, bv_m, bo_m, beta2):
    small = pl.BlockSpec((C, S), lambda i: (0, 0))
    big = pl.BlockSpec((C, C), lambda i: (0, 0))
    return pl.pallas_call(
        _tc2_body,
        grid=(G,),
        in_specs=[
            pl.BlockSpec((R, S), lambda i: (i, 0)),
            pl.BlockSpec((R, S), lambda i: (i, 0)),
            pl.BlockSpec((R, S), lambda i: (i + OUT_STRIDE // R, 0)),
            small,
            pl.BlockSpec((8, S), lambda i: (0, 0)),
            big, big, big, big,
            small, small, small, small,
            pl.BlockSpec((1, 1), lambda i: (0, 0)),
        ],
        out_specs=pl.BlockSpec((C, S), lambda i: (0, 0)),
        out_shape=jax.ShapeDtypeStruct((C, S), jnp.float32),
        scratch_shapes=[
            pltpu.VMEM((S, S), jnp.float32),
            pltpu.VMEM((C, S), jnp.float32),
        ],
    )(weights, g0, g1, xtw, wsum, wq, wk, wv, wo,
      bq_m, bk_m, bv_m, bo_m, beta2)


def _tc3_body(w_ref, so_ref, out_ref):
    out_ref[...] = lax.dot_general(
        w_ref[...], so_ref[...], (((1,), (1,)), ((), ())),
        preferred_element_type=jnp.float32)


def _tc3(weights, so_t):
    return pl.pallas_call(
        _tc3_body,
        grid=(G,),
        in_specs=[
            pl.BlockSpec((R, S), lambda i: (i, 0)),
            pl.BlockSpec((C, S), lambda i: (0, 0)),
        ],
        out_specs=pl.BlockSpec((R, C), lambda i: (i, 0)),
        out_shape=jax.ShapeDtypeStruct((N, C), jnp.float32),
    )(weights, so_t)


def kernel(x, adj_indices, adj_values, Wslice, bslice, Wq, bq, Wk, bk,
           Wv, bv, Wo, bo, beta_raw):
    xs = x[0]
    wst = Wslice.T
    bs_row = bslice.reshape(1, S)

    weights, xtw, wsum = _tc1(xs, wst, bs_row)

    dst = adj_indices[0]
    src = adj_indices[1]
    pad = EPAD - E
    zi = jnp.zeros((pad,), jnp.int32)
    srcp = jnp.concatenate([src, zi]).reshape(NW, NB, BE)
    dstp = jnp.concatenate([dst, zi]).reshape(NW, NB, BE)
    valp = jnp.concatenate([adj_values,
                            jnp.zeros((pad,), jnp.float32)]).reshape(NW, NB, BE)
    zeros_t = jnp.zeros((N, S), jnp.float32)

    gpart = _sc_agg(weights, srcp, dstp, valp, zeros_t)

    bq_m = jnp.broadcast_to(bq.reshape(C, 1), (C, S))
    bk_m = jnp.broadcast_to(bk.reshape(C, 1), (C, S))
    bv_m = jnp.broadcast_to(bv.reshape(C, 1), (C, S))
    bo_m = jnp.broadcast_to(bo.reshape(C, 1), (C, S))
    beta2 = beta_raw.reshape(1, 1)

    so_t = _tc2(weights, gpart, gpart, xtw, wsum, Wq, Wk, Wv, Wo,
                bq_m, bk_m, bv_m, bo_m, beta2)

    out = _tc3(weights, so_t)
    return out.reshape(1, N, C)


# speedup vs baseline: 1.0002x; 1.0002x over previous
import functools
import math

import jax
import jax.numpy as jnp
from jax import lax
from jax.experimental import pallas as pl
from jax.experimental.pallas import tpu as pltpu
from jax.experimental.pallas import tpu_sc as plsc

N = 10000
C = 256
S = 64
H = 8
D = C // H
E = 160000
EPS = 1e-6

NC = 2
NS = 16
LANES = 16
NW = NC * NS
BE = 128
EPT = 5120
NB = EPT // BE
EPAD = NW * EPT
NBUF = 4
ZTILES = 10
ZROWS = N // ZTILES
OUT_STRIDE = 16000

R = 2000
G = N // R


def _tc1_body(x_ref, wst_ref, bs_ref, w_ref, xtw_ref, wsum_ref):
    i = pl.program_id(0)
    xb = x_ref[...]
    logits = lax.dot_general(xb, wst_ref[...], (((1,), (0,)), ((), ())),
                             preferred_element_type=jnp.float32)
    logits = logits + bs_ref[...]
    m = jnp.max(logits, axis=1, keepdims=True)
    ew = jnp.exp(logits - m)
    w = ew / jnp.sum(ew, axis=1, keepdims=True)
    w_ref[...] = w
    xtw = lax.dot_general(xb, w, (((0,), (0,)), ((), ())),
                          preferred_element_type=jnp.float32)
    ws = jnp.broadcast_to(jnp.sum(w, axis=0, keepdims=True), (8, S))

    @pl.when(i == 0)
    def _init():
        xtw_ref[...] = jnp.zeros_like(xtw_ref)
        wsum_ref[...] = jnp.zeros_like(wsum_ref)

    xtw_ref[...] += xtw
    wsum_ref[...] += ws


def _tc1(xs, wst, bs_row):
    return pl.pallas_call(
        _tc1_body,
        grid=(G,),
        in_specs=[
            pl.BlockSpec((R, C), lambda i: (i, 0)),
            pl.BlockSpec((C, S), lambda i: (0, 0)),
            pl.BlockSpec((1, S), lambda i: (0, 0)),
        ],
        out_specs=[
            pl.BlockSpec((R, S), lambda i: (i, 0)),
            pl.BlockSpec((C, S), lambda i: (0, 0)),
            pl.BlockSpec((8, S), lambda i: (0, 0)),
        ],
        out_shape=[
            jax.ShapeDtypeStruct((N, S), jnp.float32),
            jax.ShapeDtypeStruct((C, S), jnp.float32),
            jax.ShapeDtypeStruct((8, S), jnp.float32),
        ],
    )(xs, wst, bs_row)


def _sc_agg_body(w_hbm, src_hbm, dst_hbm, val_hbm, zro_hbm, out_hbm,
                 src_v, dst_v, val_v, rows0, rows1, rows2, rows3, acc_sh,
                 sg0, sg1, sg2, sg3, ss0, ss1, ss2, ss3):
    cid = lax.axis_index("c")
    sid = lax.axis_index("s")
    wid = cid * NS + sid
    bufs = ((rows0, sg0, ss0), (rows1, sg1, ss1),
            (rows2, sg2, ss2), (rows3, sg3, ss3))

    pltpu.sync_copy(src_hbm.at[wid], src_v)
    pltpu.sync_copy(dst_hbm.at[wid], dst_v)
    pltpu.sync_copy(val_hbm.at[wid], val_v)
    pltpu.async_copy(w_hbm.at[src_v.at[0]], rows0, sg0)
    pltpu.async_copy(w_hbm.at[src_v.at[1]], rows1, sg1)
    @pl.when(sid < ZTILES)
    def _zero():
        pltpu.sync_copy(zro_hbm.at[pl.ds(sid * ZROWS, ZROWS)],
                        acc_sh.at[pl.ds(sid * ZROWS, ZROWS)])

    plsc.subcore_barrier()

    def pair(q, carry):
        for b in range(2):
            j = 2 * q + b
            rows_v, sg, ss = bufs[b]
            pltpu.make_async_copy(w_hbm.at[src_v.at[j]], rows_v, sg).wait()

            @plsc.parallel_loop(0, BE // LANES, 1, unroll=2)
            def _scale(g):
                vals16 = val_v[j, pl.ds(g * LANES, LANES)]
                for el in range(LANES):
                    v = vals16[el]
                    e = g * LANES + el
                    for cc in range(S // LANES):
                        sl = pl.ds(cc * LANES, LANES)
                        rows_v[e, sl] = rows_v[e, sl] * v

            pltpu.sync_copy(rows_v, acc_sh.at[dst_v.at[j]], add=True)

            @pl.when(j + 2 < NB)
            def _refill():
                pltpu.async_copy(w_hbm.at[src_v.at[j + 2]], rows_v, sg)
        return carry

    lax.fori_loop(0, NB // 2, pair, 0)
    plsc.subcore_barrier()
    @pl.when(sid < ZTILES)
    def _out():
        pltpu.sync_copy(acc_sh.at[pl.ds(sid * ZROWS, ZROWS)],
                        out_hbm.at[pl.ds(cid * OUT_STRIDE + sid * ZROWS,
                                         ZROWS)])


def _sc_agg(weights, srcp, dstp, valp, zeros_t):
    mesh = plsc.VectorSubcoreMesh(core_axis_name="c", subcore_axis_name="s")
    f = functools.partial(
        pl.kernel,
        mesh=mesh,
        compiler_params=pltpu.CompilerParams(use_tc_tiling_on_sc=False),
        out_type=jax.ShapeDtypeStruct((NC * OUT_STRIDE, S), jnp.float32),
        scratch_types=(
            [pltpu.VMEM((NB, BE), jnp.int32),
             pltpu.VMEM((NB, BE), jnp.int32),
             pltpu.VMEM((NB, BE), jnp.float32)]
            + [pltpu.VMEM((BE, S), jnp.float32)] * NBUF
            + [pltpu.VMEM_SHARED((N, S), jnp.float32)]
            + [pltpu.SemaphoreType.DMA] * (2 * NBUF)
        ),
    )(_sc_agg_body)
    return f(weights, srcp, dstp, valp, zeros_t)


def _tc2_body(w_ref, g0_ref, g1_ref, xtw_ref, wsum_ref,
              wq_ref, wk_ref, wv_ref, wo_ref,
              bq_ref, bk_ref, bv_ref, bo_ref, beta_ref,
              so_ref, bias_acc, attn_acc):
    i = pl.program_id(0)
    w = w_ref[...]
    g = g0_ref[...] + g1_ref[...]
    part = lax.dot_general(w, g, (((0,), (0,)), ((), ())),
                           preferred_element_type=jnp.float32)

    @pl.when(i == 0)
    def _init():
        bias_acc[...] = jnp.zeros_like(bias_acc)

    bias_acc[...] += part

    @pl.when(i == G - 1)
    def _finish():
        wsum = jnp.maximum(wsum_ref[0:1, :], EPS)
        slices_t = xtw_ref[...] / wsum
        qt = lax.dot_general(wq_ref[...], slices_t, (((1,), (0,)), ((), ())),
                             preferred_element_type=jnp.float32) + bq_ref[...]
        kt = lax.dot_general(wk_ref[...], slices_t, (((1,), (0,)), ((), ())),
                             preferred_element_type=jnp.float32) + bk_ref[...]
        vt = lax.dot_general(wv_ref[...], slices_t, (((1,), (0,)), ((), ())),
                             preferred_element_type=jnp.float32) + bv_ref[...]

        bias = bias_acc[...]
        ii = lax.broadcasted_iota(jnp.int32, (S, S), 0)
        jj = lax.broadcasted_iota(jnp.int32, (S, S), 1)
        eye = (ii == jj).astype(jnp.float32)
        bias_tr = lax.dot_general(bias, eye, (((0,), (0,)), ((), ())),
                                  preferred_element_type=jnp.float32)
        gb = 0.5 * (bias + bias_tr)
        gb = gb / jnp.maximum(jnp.sum(gb, axis=1, keepdims=True), EPS)
        gb = jnp.log(jnp.maximum(gb, EPS))
        br = beta_ref[0, 0]
        beta = jnp.maximum(br, 0.0) + jnp.log(1.0 + jnp.exp(-jnp.abs(br)))
        gbias = beta * gb
        scale = 1.0 / math.sqrt(D)

        for h in range(H):
            qh = qt[h * D:(h + 1) * D, :]
            kh = kt[h * D:(h + 1) * D, :]
            vh = vt[h * D:(h + 1) * D, :]
            lg = lax.dot_general(qh, kh, (((0,), (0,)), ((), ())),
                                 preferred_element_type=jnp.float32)
            lg = lg * scale + gbias
            m = jnp.max(lg, axis=1, keepdims=True)
            a = jnp.exp(lg - m)
            a = a / jnp.sum(a, axis=1, keepdims=True)
            oh = lax.dot_general(vh, a, (((1,), (1,)), ((), ())),
                                 preferred_element_type=jnp.float32)
            attn_acc[h * D:(h + 1) * D, :] = oh

        so_ref[...] = lax.dot_general(
            wo_ref[...], attn_acc[...], (((1,), (0,)), ((), ())),
            preferred_element_type=jnp.float32) + bo_ref[...]


def _tc2(weights, g0, g1, xtw, wsum, wq, wk, wv, wo,
         bq_m, bk_m, bv_m, bo_m, beta2):
    small = pl.BlockSpec((C, S), lambda i: (0, 0))
    big = pl.BlockSpec((C, C), lambda i: (0, 0))
    return pl.pallas_call(
        _tc2_body,
        grid=(G,),
        in_specs=[
            pl.BlockSpec((R, S), lambda i: (i, 0)),
            pl.BlockSpec((R, S), lambda i: (i, 0)),
            pl.BlockSpec((R, S), lambda i: (i + OUT_STRIDE // R, 0)),
            small,
            pl.BlockSpec((8, S), lambda i: (0, 0)),
            big, big, big, big,
            small, small, small, small,
            pl.BlockSpec((1, 1), lambda i: (0, 0)),
        ],
        out_specs=pl.BlockSpec((C, S), lambda i: (0, 0)),
        out_shape=jax.ShapeDtypeStruct((C, S), jnp.float32),
        scratch_shapes=[
            pltpu.VMEM((S, S), jnp.float32),
            pltpu.VMEM((C, S), jnp.float32),
        ],
    )(weights, g0, g1, xtw, wsum, wq, wk, wv, wo,
      bq_m, bk_m, bv_m, bo_m, beta2)


def _tc3_body(w_ref, so_ref, out_ref):
    out_ref[...] = lax.dot_general(
        w_ref[...], so_ref[...], (((1,), (1,)), ((), ())),
        preferred_element_type=jnp.float32)


def _tc3(weights, so_t):
    return pl.pallas_call(
        _tc3_body,
        grid=(G,),
        in_specs=[
            pl.BlockSpec((R, S), lambda i: (i, 0)),
            pl.BlockSpec((C, S), lambda i: (0, 0)),
        ],
        out_specs=pl.BlockSpec((R, C), lambda i: (i, 0)),
        out_shape=jax.ShapeDtypeStruct((N, C), jnp.float32),
    )(weights, so_t)


def kernel(x, adj_indices, adj_values, Wslice, bslice, Wq, bq, Wk, bk,
           Wv, bv, Wo, bo, beta_raw):
    xs = x[0]
    wst = Wslice.T
    bs_row = bslice.reshape(1, S)

    weights, xtw, wsum = _tc1(xs, wst, bs_row)

    dst = adj_indices[0]
    src = adj_indices[1]
    pad = EPAD - E
    zi = jnp.zeros((pad,), jnp.int32)
    srcp = jnp.concatenate([src, zi]).reshape(NW, NB, BE)
    dstp = jnp.concatenate([dst, zi]).reshape(NW, NB, BE)
    valp = jnp.concatenate([adj_values,
                            jnp.zeros((pad,), jnp.float32)]).reshape(NW, NB, BE)
    zeros_t = jnp.zeros((N, S), jnp.float32)

    gpart = _sc_agg(weights, srcp, dstp, valp, zeros_t)

    bq_m = jnp.broadcast_to(bq.reshape(C, 1), (C, S))
    bk_m = jnp.broadcast_to(bk.reshape(C, 1), (C, S))
    bv_m = jnp.broadcast_to(bv.reshape(C, 1), (C, S))
    bo_m = jnp.broadcast_to(bo.reshape(C, 1), (C, S))
    beta2 = beta_raw.reshape(1, 1)

    so_t = _tc2(weights, gpart, gpart, xtw, wsum, Wq, Wk, Wv, Wo,
                bq_m, bk_m, bv_m, bo_m, beta2)

    out = _tc3(weights, so_t)
    return out.reshape(1, N, C)


# speedup vs baseline: 1.2011x; 1.2009x over previous
import functools
import math

import jax
import jax.numpy as jnp
from jax import lax
from jax.experimental import pallas as pl
from jax.experimental.pallas import tpu as pltpu
from jax.experimental.pallas import tpu_sc as plsc

N = 10000
C = 256
S = 64
H = 8
D = C // H
E = 160000
EPS = 1e-6

NC = 2
NS = 16
LANES = 16
NW = NC * NS
BE = 128
EPT = 5120
NB = EPT // BE
EPAD = NW * EPT
ACC_N = 10240
RPT = ACC_N // NS

R = 2000
G = N // R


def _tc1_body(x_ref, wst_ref, bs_ref, w_ref, xtw_ref, wsum_ref):
    i = pl.program_id(0)
    xb = x_ref[...]
    logits = lax.dot_general(xb, wst_ref[...], (((1,), (0,)), ((), ())),
                             preferred_element_type=jnp.float32)
    logits = logits + bs_ref[...]
    m = jnp.max(logits, axis=1, keepdims=True)
    ew = jnp.exp(logits - m)
    w = ew / jnp.sum(ew, axis=1, keepdims=True)
    w_ref[...] = w
    xtw = lax.dot_general(xb, w, (((0,), (0,)), ((), ())),
                          preferred_element_type=jnp.float32)
    ws = jnp.broadcast_to(jnp.sum(w, axis=0, keepdims=True), (8, S))

    @pl.when(i == 0)
    def _init():
        xtw_ref[...] = jnp.zeros_like(xtw_ref)
        wsum_ref[...] = jnp.zeros_like(wsum_ref)

    xtw_ref[...] += xtw
    wsum_ref[...] += ws


def _tc1(xs, wst, bs_row):
    return pl.pallas_call(
        _tc1_body,
        grid=(G,),
        in_specs=[
            pl.BlockSpec((R, C), lambda i: (i, 0)),
            pl.BlockSpec((C, S), lambda i: (0, 0)),
            pl.BlockSpec((1, S), lambda i: (0, 0)),
        ],
        out_specs=[
            pl.BlockSpec((R, S), lambda i: (i, 0)),
            pl.BlockSpec((C, S), lambda i: (0, 0)),
            pl.BlockSpec((8, S), lambda i: (0, 0)),
        ],
        out_shape=[
            jax.ShapeDtypeStruct((N, S), jnp.float32),
            jax.ShapeDtypeStruct((C, S), jnp.float32),
            jax.ShapeDtypeStruct((8, S), jnp.float32),
        ],
    )(xs, wst, bs_row)


def _sc_agg_body(w_hbm, src_hbm, dst_hbm, val_hbm, zro_hbm, out_hbm,
                 src_v, dst_v, val_v, rows0, rows1, acc_sh, sg0, sg1):
    cid = lax.axis_index("c")
    sid = lax.axis_index("s")
    wid = cid * NS + sid
    bufs = ((rows0, sg0), (rows1, sg1))

    pltpu.sync_copy(src_hbm.at[wid], src_v)
    pltpu.sync_copy(dst_hbm.at[wid], dst_v)
    pltpu.sync_copy(val_hbm.at[wid], val_v)
    pltpu.async_copy(w_hbm.at[src_v.at[0]], rows0, sg0)
    pltpu.async_copy(w_hbm.at[src_v.at[1]], rows1, sg1)
    pltpu.sync_copy(zro_hbm.at[pl.ds(sid * RPT, RPT)],
                    acc_sh.at[pl.ds(sid * RPT, RPT)])
    plsc.subcore_barrier()

    def pair(q, carry):
        for b in range(2):
            j = 2 * q + b
            rows_v, sg = bufs[b]
            pltpu.make_async_copy(w_hbm.at[src_v.at[j]], rows_v, sg).wait()

            @plsc.parallel_loop(0, BE // LANES, 1, unroll=2)
            def _scale(g):
                vals16 = val_v[j, pl.ds(g * LANES, LANES)]
                for el in range(LANES):
                    v = vals16[el]
                    e = g * LANES + el
                    for cc in range(S // LANES):
                        sl = pl.ds(cc * LANES, LANES)
                        rows_v[e, sl] = rows_v[e, sl] * v

            pltpu.sync_copy(rows_v, acc_sh.at[dst_v.at[j]], add=True)

            @pl.when(j + 2 < NB)
            def _refill():
                pltpu.async_copy(w_hbm.at[src_v.at[j + 2]], rows_v, sg)
        return carry

    lax.fori_loop(0, NB // 2, pair, 0)
    plsc.subcore_barrier()
    pltpu.sync_copy(acc_sh.at[pl.ds(sid * RPT, RPT)],
                    out_hbm.at[pl.ds(cid * ACC_N + sid * RPT, RPT)])


def _sc_agg(weights, srcp, dstp, valp, zeros_t):
    mesh = plsc.VectorSubcoreMesh(core_axis_name="c", subcore_axis_name="s")
    f = functools.partial(
        pl.kernel,
        mesh=mesh,
        compiler_params=pltpu.CompilerParams(use_tc_tiling_on_sc=False),
        out_type=jax.ShapeDtypeStruct((NC * ACC_N, S), jnp.float32),
        scratch_types=[
            pltpu.VMEM((NB, BE), jnp.int32),
            pltpu.VMEM((NB, BE), jnp.int32),
            pltpu.VMEM((NB, BE), jnp.float32),
            pltpu.VMEM((BE, S), jnp.float32),
            pltpu.VMEM((BE, S), jnp.float32),
            pltpu.VMEM_SHARED((ACC_N, S), jnp.float32),
            pltpu.SemaphoreType.DMA,
            pltpu.SemaphoreType.DMA,
        ],
    )(_sc_agg_body)
    return f(weights, srcp, dstp, valp, zeros_t)


def _tc2_body(w_ref, g0_ref, g1_ref, xtw_ref, wsum_ref,
              wq_ref, wk_ref, wv_ref, wo_ref,
              bq_ref, bk_ref, bv_ref, bo_ref, beta_ref,
              so_ref, bias_acc, attn_acc):
    i = pl.program_id(0)
    w = w_ref[...]
    g = g0_ref[...] + g1_ref[...]
    part = lax.dot_general(w, g, (((0,), (0,)), ((), ())),
                           preferred_element_type=jnp.float32)

    @pl.when(i == 0)
    def _init():
        bias_acc[...] = jnp.zeros_like(bias_acc)

    bias_acc[...] += part

    @pl.when(i == G - 1)
    def _finish():
        wsum = jnp.maximum(wsum_ref[0:1, :], EPS)
        slices_t = xtw_ref[...] / wsum
        qt = lax.dot_general(wq_ref[...], slices_t, (((1,), (0,)), ((), ())),
                             preferred_element_type=jnp.float32) + bq_ref[...]
        kt = lax.dot_general(wk_ref[...], slices_t, (((1,), (0,)), ((), ())),
                             preferred_element_type=jnp.float32) + bk_ref[...]
        vt = lax.dot_general(wv_ref[...], slices_t, (((1,), (0,)), ((), ())),
                             preferred_element_type=jnp.float32) + bv_ref[...]

        bias = bias_acc[...]
        ii = lax.broadcasted_iota(jnp.int32, (S, S), 0)
        jj = lax.broadcasted_iota(jnp.int32, (S, S), 1)
        eye = (ii == jj).astype(jnp.float32)
        bias_tr = lax.dot_general(bias, eye, (((0,), (0,)), ((), ())),
                                  preferred_element_type=jnp.float32)
        gb = 0.5 * (bias + bias_tr)
        gb = gb / jnp.maximum(jnp.sum(gb, axis=1, keepdims=True), EPS)
        gb = jnp.log(jnp.maximum(gb, EPS))
        br = beta_ref[0, 0]
        beta = jnp.maximum(br, 0.0) + jnp.log(1.0 + jnp.exp(-jnp.abs(br)))
        gbias = beta * gb
        scale = 1.0 / math.sqrt(D)

        for h in range(H):
            qh = qt[h * D:(h + 1) * D, :]
            kh = kt[h * D:(h + 1) * D, :]
            vh = vt[h * D:(h + 1) * D, :]
            lg = lax.dot_general(qh, kh, (((0,), (0,)), ((), ())),
                                 preferred_element_type=jnp.float32)
            lg = lg * scale + gbias
            m = jnp.max(lg, axis=1, keepdims=True)
            a = jnp.exp(lg - m)
            a = a / jnp.sum(a, axis=1, keepdims=True)
            oh = lax.dot_general(vh, a, (((1,), (1,)), ((), ())),
                                 preferred_element_type=jnp.float32)
            attn_acc[h * D:(h + 1) * D, :] = oh

        so_ref[...] = lax.dot_general(
            wo_ref[...], attn_acc[...], (((1,), (0,)), ((), ())),
            preferred_element_type=jnp.float32) + bo_ref[...]


def _tc2(weights, g0, g1, xtw, wsum, wq, wk, wv, wo,
         bq_m, bk_m, bv_m, bo_m, beta2):
    small = pl.BlockSpec((C, S), lambda i: (0, 0))
    big = pl.BlockSpec((C, C), lambda i: (0, 0))
    return pl.pallas_call(
        _tc2_body,
        grid=(G,),
        in_specs=[
            pl.BlockSpec((R, S), lambda i: (i, 0)),
            pl.BlockSpec((R, S), lambda i: (i, 0)),
            pl.BlockSpec((R, S), lambda i: (i, 0)),
            small,
            pl.BlockSpec((8, S), lambda i: (0, 0)),
            big, big, big, big,
            small, small, small, small,
            pl.BlockSpec((1, 1), lambda i: (0, 0)),
        ],
        out_specs=pl.BlockSpec((C, S), lambda i: (0, 0)),
        out_shape=jax.ShapeDtypeStruct((C, S), jnp.float32),
        scratch_shapes=[
            pltpu.VMEM((S, S), jnp.float32),
            pltpu.VMEM((C, S), jnp.float32),
        ],
    )(weights, g0, g1, xtw, wsum, wq, wk, wv, wo,
      bq_m, bk_m, bv_m, bo_m, beta2)


def _tc3_body(w_ref, so_ref, out_ref):
    out_ref[...] = lax.dot_general(
        w_ref[...], so_ref[...], (((1,), (1,)), ((), ())),
        preferred_element_type=jnp.float32)


def _tc3(weights, so_t):
    return pl.pallas_call(
        _tc3_body,
        grid=(G,),
        in_specs=[
            pl.BlockSpec((R, S), lambda i: (i, 0)),
            pl.BlockSpec((C, S), lambda i: (0, 0)),
        ],
        out_specs=pl.BlockSpec((R, C), lambda i: (i, 0)),
        out_shape=jax.ShapeDtypeStruct((N, C), jnp.float32),
    )(weights, so_t)


def kernel(x, adj_indices, adj_values, Wslice, bslice, Wq, bq, Wk, bk,
           Wv, bv, Wo, bo, beta_raw):
    xs = x[0]
    wst = Wslice.T
    bs_row = bslice.reshape(1, S)

    weights, xtw, wsum = _tc1(xs, wst, bs_row)

    dst = adj_indices[0]
    src = adj_indices[1]
    pad = EPAD - E
    zi = jnp.zeros((pad,), jnp.int32)
    srcp = jnp.concatenate([src, zi]).reshape(NW, NB, BE)
    dstp = jnp.concatenate([dst, zi]).reshape(NW, NB, BE)
    valp = jnp.concatenate([adj_values,
                            jnp.zeros((pad,), jnp.float32)]).reshape(NW, NB, BE)
    zeros_t = jnp.zeros((ACC_N, S), jnp.float32)

    gpart = _sc_agg(weights, srcp, dstp, valp, zeros_t)
    g0 = gpart[:N]
    g1 = gpart[ACC_N:ACC_N + N]

    bq_m = jnp.broadcast_to(bq.reshape(C, 1), (C, S))
    bk_m = jnp.broadcast_to(bk.reshape(C, 1), (C, S))
    bv_m = jnp.broadcast_to(bv.reshape(C, 1), (C, S))
    bo_m = jnp.broadcast_to(bo.reshape(C, 1), (C, S))
    beta2 = beta_raw.reshape(1, 1)

    so_t = _tc2(weights, g0, g1, xtw, wsum, Wq, Wk, Wv, Wo,
                bq_m, bk_m, bv_m, bo_m, beta2)

    out = _tc3(weights, so_t)
    return out.reshape(1, N, C)
